# baseline (speedup 1.0000x reference)
import functools

import jax
import jax.numpy as jnp
from jax import lax
from jax.experimental import pallas as pl
from jax.experimental.pallas import tpu as pltpu
from jax.experimental.pallas import tpu_sc as plsc

MAXLEN = 200
EMBED = 32
VOCAB = 1000000
BATCH = 4096
ROWS = BATCH * MAXLEN

NC, NS = 2, 16
NW = NC * NS

NBLK = VOCAB // 128
TAIL = VOCAB - NBLK * 128
K1_EXTRA = NBLK - (NBLK // NW) * NW
K1_CNT = NBLK // NW


def _k1_body(tt_ref, tail_ref, rm_ref, bin_v, bout_v, sem):
    wid = lax.axis_index("s") * NC + lax.axis_index("c")
    cnt = jnp.where(wid < K1_EXTRA, K1_CNT + 1, K1_CNT)
    start = wid * K1_CNT + jnp.minimum(wid, K1_EXTRA)
    iota = lax.iota(jnp.int32, 16)

    def do_block(i, b):
        @pl.when(i < cnt)
        def _():
            j = start + i
            pltpu.sync_copy(tt_ref.at[:, pl.ds(j * 128, 128)], bin_v.at[b])

            def tstep(t, acc):
                row = t // 4
                col = (t % 4) * EMBED
                for h in range(2):
                    vec = plsc.load_gather(
                        bin_v.at[b],
                        [iota + (h * 16), jnp.broadcast_to(t, (16,))],
                    )
                    bout_v[b, row, pl.ds(col + h * 16, 16)] = vec
                return acc

            lax.fori_loop(0, 128, tstep, 0)
            pltpu.sync_copy(bout_v.at[b], rm_ref.at[pl.ds(j * 32, 32)])

    def pair(k, carry):
        do_block(2 * k, 0)
        do_block(2 * k + 1, 1)
        return carry

    lax.fori_loop(0, (K1_CNT + 2) // 2, pair, 0)

    @pl.when(wid == NW - 1)
    def _():
        pltpu.sync_copy(tail_ref, rm_ref.at[pl.ds(NBLK * 32, TAIL * EMBED // 128)])


K2_UNITS = ROWS // 1024
K2_PER_W = K2_UNITS // NW


def _k2_body(valt_ref, rm_ref, pat_ref, mid_ref, idx_v, buf_v, pat_v, sem, osem0, osem1):
    wid = lax.axis_index("s") * NC + lax.axis_index("c")
    pltpu.sync_copy(pat_ref, pat_v)
    base_u = wid * K2_PER_W

    def do_unit(i, b, osem):
        u = base_u + i
        s = u // 4
        prow = s // 4
        pcol = (s % 4) * EMBED
        pltpu.sync_copy(valt_ref.at[pl.ds(u * 8, 8)], idx_v.at[b])
        copies = []
        for j in range(8):
            copies.append(
                pltpu.async_copy(
                    rm_ref.at[idx_v.at[b, j]],
                    buf_v.at[b, pl.ds(j * 128, 128)],
                    sem,
                )
            )
        for cp in copies:
            cp.wait()
        pv0 = pat_v[prow, pl.ds(pcol, 16)]
        pv1 = pat_v[prow, pl.ds(pcol + 16, 16)]

        def add_step(r, acc):
            plsc.addupdate(buf_v.at[b, r, pl.ds(0, 16)], pv0)
            plsc.addupdate(buf_v.at[b, r, pl.ds(16, 16)], pv1)
            return acc

        lax.fori_loop(0, 1024, add_step, 0)
        return pltpu.async_copy(buf_v.at[b], mid_ref.at[pl.ds(u * 1024, 1024)], osem)

    sems = [osem0, osem1]
    handles = [None, None]
    for i in range(K2_PER_W):
        b = i % 2
        if handles[b] is not None:
            handles[b].wait()
        handles[b] = do_unit(i, b, sems[b])
    handles[0].wait()
    handles[1].wait()


K3_UNITS = MAXLEN * (BATCH // 128)
K3_PER_W = K3_UNITS // NW


def _k3_body(mid_ref, out_ref, bin_v, bout_v, sem):
    wid = lax.axis_index("s") * NC + lax.axis_index("c")
    base_u = wid * K3_PER_W
    iota = lax.iota(jnp.int32, 16)

    def do_unit(i, b):
        u = base_u + i
        s = u // 32
        jcol = (u % 32) * 128
        pltpu.sync_copy(mid_ref.at[pl.ds(u * 32, 32)], bin_v.at[b])

        def tstep(t, acc):
            row = t // 4
            col = (t % 4) * EMBED
            for h in range(2):
                vec = bin_v[b, row, pl.ds(col + h * 16, 16)]
                plsc.store_scatter(
                    bout_v.at[b],
                    [iota + (h * 16), jnp.broadcast_to(t, (16,))],
                    vec,
                )
            return acc

        lax.fori_loop(0, 128, tstep, 0)
        pltpu.sync_copy(bout_v.at[b], out_ref.at[s, :, pl.ds(jcol, 128)])

    def pair(k, carry):
        do_unit(2 * k, 0)
        do_unit(2 * k + 1, 1)
        return carry

    lax.fori_loop(0, K3_PER_W // 2, pair, 0)


_MESH = plsc.VectorSubcoreMesh(core_axis_name="c", subcore_axis_name="s")


@jax.jit
def _run(val, token_table, pos_table):
    tt = token_table.T
    tail = token_table[NBLK * 128:].reshape(TAIL * EMBED // 128, 128)
    valt = val.T.astype(jnp.int32).reshape(ROWS // 128, 128)
    patq = pos_table.reshape(MAXLEN * EMBED // 128, 128)

    k1 = functools.partial(
        pl.kernel,
        mesh=_MESH,
        out_type=jax.ShapeDtypeStruct((VOCAB * EMBED // 128, 128), jnp.float32),
        scratch_types=[
            pltpu.VMEM((2, EMBED, 128), jnp.float32),
            pltpu.VMEM((2, 32, 128), jnp.float32),
            pltpu.SemaphoreType.DMA,
        ],
        compiler_params=pltpu.CompilerParams(needs_layout_passes=False),
    )(_k1_body)
    rm4 = k1(tt, tail)

    k2 = functools.partial(
        pl.kernel,
        mesh=_MESH,
        out_type=jax.ShapeDtypeStruct((ROWS, EMBED), jnp.float32),
        scratch_types=[
            pltpu.VMEM((2, 8, 128), jnp.int32),
            pltpu.VMEM((2, 1024, EMBED), jnp.float32),
            pltpu.VMEM((MAXLEN * EMBED // 128, 128), jnp.float32),
            pltpu.SemaphoreType.DMA,
            pltpu.SemaphoreType.DMA,
            pltpu.SemaphoreType.DMA,
        ],
        compiler_params=pltpu.CompilerParams(use_tc_tiling_on_sc=False),
    )(_k2_body)
    mid = k2(valt, rm4.reshape(VOCAB, EMBED), patq)

    k3 = functools.partial(
        pl.kernel,
        mesh=_MESH,
        out_type=jax.ShapeDtypeStruct((MAXLEN, EMBED, BATCH), jnp.float32),
        scratch_types=[
            pltpu.VMEM((2, 32, 128), jnp.float32),
            pltpu.VMEM((2, EMBED, 128), jnp.float32),
            pltpu.SemaphoreType.DMA,
        ],
        compiler_params=pltpu.CompilerParams(needs_layout_passes=False),
    )(_k3_body)
    outt = k3(mid.reshape(ROWS * EMBED // 128, 128))
    return jnp.transpose(outt, (2, 0, 1))


def kernel(val, token_table, pos_table):
    return _run(val, token_table, pos_table)


# speedup vs baseline: 1.3058x; 1.3058x over previous
import functools

import jax
import jax.numpy as jnp
from jax import lax
from jax.experimental import pallas as pl
from jax.experimental.pallas import tpu as pltpu
from jax.experimental.pallas import tpu_sc as plsc

MAXLEN = 200
EMBED = 32
VOCAB = 1000000
BATCH = 4096
ROWS = BATCH * MAXLEN

NC, NS = 2, 16
NW = NC * NS

NBLK = VOCAB // 128
TAIL = VOCAB - NBLK * 128
K1_EXTRA = NBLK - (NBLK // NW) * NW
K1_CNT = NBLK // NW


PITCH = 129


def _k1_body(tt_ref, tail_ref, rm_ref, bin_v, bout_v, isem0, isem1, ssem0, ssem1):
    wid = lax.axis_index("s") * NC + lax.axis_index("c")
    cnt = jnp.where(wid < K1_EXTRA, K1_CNT + 1, K1_CNT)
    start = wid * K1_CNT + jnp.minimum(wid, K1_EXTRA)
    iota = lax.iota(jnp.int32, 16)
    isems = (isem0, isem1)
    ssems = (ssem0, ssem1)

    def do_block(i, b):
        @pl.when(i < cnt)
        def _():
            j = start + i
            src = tt_ref.at[:, pl.ds(j * 128, 128)]
            dst = rm_ref.at[pl.ds(j * 32, 32)]
            bpad = bin_v.at[b, :, pl.ds(0, 128)]

            @pl.when(i >= 2)
            def _():
                pltpu.make_async_copy(bout_v.at[b], dst, ssems[b]).wait()
                pltpu.make_async_copy(src, bpad, isems[b]).wait()

            @pl.when(i < 2)
            def _():
                pltpu.sync_copy(src, bpad)

            def tstep(q, acc):
                for tt in range(4):
                    t = q * 4 + tt
                    bc = jnp.broadcast_to(t, (16,))
                    for h in range(2):
                        vec = plsc.load_gather(
                            bin_v.at[b], [iota + (h * 16), bc]
                        )
                        bout_v[b, q, pl.ds(tt * EMBED + h * 16, 16)] = vec
                return acc

            lax.fori_loop(0, 32, tstep, 0)

            @pl.when(i + 2 < cnt)
            def _():
                nsrc = tt_ref.at[:, pl.ds((start + i + 2) * 128, 128)]
                pltpu.async_copy(nsrc, bpad, isems[b])

            pltpu.async_copy(bout_v.at[b], dst, ssems[b])

    def pair(k, carry):
        do_block(2 * k, 0)
        do_block(2 * k + 1, 1)
        return carry

    lax.fori_loop(0, (K1_CNT + 2) // 2, pair, 0)
    for b in range(2):
        pltpu.make_async_copy(
            bout_v.at[b], rm_ref.at[pl.ds(0, 32)], ssems[b]
        ).wait()

    @pl.when(wid == NW - 1)
    def _():
        pltpu.sync_copy(tail_ref, rm_ref.at[pl.ds(NBLK * 32, TAIL * EMBED // 128)])


K2_UNITS = ROWS // 1024
K2_PER_W = K2_UNITS // NW


def _k2_body(valt_ref, rm_ref, pat_ref, mid_ref, idx_v, buf_v, pat_v, sem, osem0, osem1):
    wid = lax.axis_index("s") * NC + lax.axis_index("c")
    pltpu.sync_copy(pat_ref, pat_v)
    base_u = wid * K2_PER_W

    def do_unit(i, b, osem):
        u = base_u + i
        s = u // 4
        prow = s // 4
        pcol = (s % 4) * EMBED
        pltpu.sync_copy(valt_ref.at[pl.ds(u * 8, 8)], idx_v.at[b])
        copies = []
        for j in range(8):
            copies.append(
                pltpu.async_copy(
                    rm_ref.at[idx_v.at[b, j]],
                    buf_v.at[b, pl.ds(j * 128, 128)],
                    sem,
                )
            )
        for cp in copies:
            cp.wait()
        pv0 = pat_v[prow, pl.ds(pcol, 16)]
        pv1 = pat_v[prow, pl.ds(pcol + 16, 16)]

        def add_step(r, acc):
            plsc.addupdate(buf_v.at[b, r, pl.ds(0, 16)], pv0)
            plsc.addupdate(buf_v.at[b, r, pl.ds(16, 16)], pv1)
            return acc

        lax.fori_loop(0, 1024, add_step, 0)
        return pltpu.async_copy(buf_v.at[b], mid_ref.at[pl.ds(u * 1024, 1024)], osem)

    sems = [osem0, osem1]
    handles = [None, None]
    for i in range(K2_PER_W):
        b = i % 2
        if handles[b] is not None:
            handles[b].wait()
        handles[b] = do_unit(i, b, sems[b])
    handles[0].wait()
    handles[1].wait()


K3_UNITS = MAXLEN * (BATCH // 512)
K3_PER_W = K3_UNITS // NW


def _k3_body(mid_ref, out_ref, bin_v, bout_v, isem0, isem1, ssem0, ssem1):
    wid = lax.axis_index("s") * NC + lax.axis_index("c")
    base_u = wid * K3_PER_W
    iota = lax.iota(jnp.int32, 16)
    isems = (isem0, isem1)
    ssems = (ssem0, ssem1)

    def unit_src(u):
        return mid_ref.at[pl.ds(u * 128, 128)]

    def do_unit(i, b):
        u = base_u + i
        s = u // 8
        jcol = (u % 8) * 512

        @pl.when(i >= 2)
        def _():
            for jj in range(4):
                pltpu.make_async_copy(
                    bout_v.at[b, jj, :, pl.ds(0, 128)],
                    out_ref.at[s, :, pl.ds(jcol + jj * 128, 128)],
                    ssems[b],
                ).wait()
            pltpu.make_async_copy(unit_src(u), bin_v.at[b], isems[b]).wait()

        @pl.when(i < 2)
        def _():
            pltpu.sync_copy(unit_src(u), bin_v.at[b])

        for jj in range(4):
            def tstep(q, acc, jj=jj):
                for tt in range(4):
                    t = q * 4 + tt
                    bc = jnp.broadcast_to(t, (16,))
                    for h in range(2):
                        vec = bin_v[b, jj * 32 + q, pl.ds(tt * EMBED + h * 16, 16)]
                        plsc.store_scatter(
                            bout_v.at[b, jj], [iota + (h * 16), bc], vec
                        )
                return acc

            lax.fori_loop(0, 32, tstep, 0)

        @pl.when(i + 2 < cnt_true)
        def _():
            pltpu.async_copy(unit_src(u + 2), bin_v.at[b], isems[b])

        for jj in range(4):
            pltpu.async_copy(
                bout_v.at[b, jj, :, pl.ds(0, 128)],
                out_ref.at[s, :, pl.ds(jcol + jj * 128, 128)],
                ssems[b],
            )

    cnt_true = K3_PER_W

    def pair(k, carry):
        do_unit(2 * k, 0)
        do_unit(2 * k + 1, 1)
        return carry

    lax.fori_loop(0, K3_PER_W // 2, pair, 0)
    for b in range(2):
        for jj in range(4):
            pltpu.make_async_copy(
                bout_v.at[b, jj, :, pl.ds(0, 128)],
                out_ref.at[0, :, pl.ds(jj * 128, 128)],
                ssems[b],
            ).wait()


_MESH = plsc.VectorSubcoreMesh(core_axis_name="c", subcore_axis_name="s")


@jax.jit
def _run(val, token_table, pos_table):
    tt = token_table.T
    tail = token_table[NBLK * 128:].reshape(TAIL * EMBED // 128, 128)
    valt = val.T.astype(jnp.int32).reshape(ROWS // 128, 128)
    patq = pos_table.reshape(MAXLEN * EMBED // 128, 128)

    k1 = functools.partial(
        pl.kernel,
        mesh=_MESH,
        out_type=jax.ShapeDtypeStruct((VOCAB * EMBED // 128, 128), jnp.float32),
        scratch_types=[
            pltpu.VMEM((2, EMBED, PITCH), jnp.float32),
            pltpu.VMEM((2, 32, 128), jnp.float32),
            pltpu.SemaphoreType.DMA,
            pltpu.SemaphoreType.DMA,
            pltpu.SemaphoreType.DMA,
            pltpu.SemaphoreType.DMA,
        ],
        compiler_params=pltpu.CompilerParams(needs_layout_passes=False),
    )(_k1_body)
    rm4 = k1(tt, tail)

    k2 = functools.partial(
        pl.kernel,
        mesh=_MESH,
        out_type=jax.ShapeDtypeStruct((ROWS, EMBED), jnp.float32),
        scratch_types=[
            pltpu.VMEM((2, 8, 128), jnp.int32),
            pltpu.VMEM((2, 1024, EMBED), jnp.float32),
            pltpu.VMEM((MAXLEN * EMBED // 128, 128), jnp.float32),
            pltpu.SemaphoreType.DMA,
            pltpu.SemaphoreType.DMA,
            pltpu.SemaphoreType.DMA,
        ],
        compiler_params=pltpu.CompilerParams(use_tc_tiling_on_sc=False),
    )(_k2_body)
    mid = k2(valt, rm4.reshape(VOCAB, EMBED), patq)

    k3 = functools.partial(
        pl.kernel,
        mesh=_MESH,
        out_type=jax.ShapeDtypeStruct((MAXLEN, EMBED, BATCH), jnp.float32),
        scratch_types=[
            pltpu.VMEM((2, 128, 128), jnp.float32),
            pltpu.VMEM((2, 4, EMBED, PITCH), jnp.float32),
            pltpu.SemaphoreType.DMA,
            pltpu.SemaphoreType.DMA,
            pltpu.SemaphoreType.DMA,
            pltpu.SemaphoreType.DMA,
        ],
        compiler_params=pltpu.CompilerParams(needs_layout_passes=False),
    )(_k3_body)
    outt = k3(mid.reshape(ROWS * EMBED // 128, 128))
    return jnp.transpose(outt, (2, 0, 1))


def kernel(val, token_table, pos_table):
    return _run(val, token_table, pos_table)


# speedup vs baseline: 1.8003x; 1.3787x over previous
import functools

import jax
import jax.numpy as jnp
from jax import lax
from jax.experimental import pallas as pl
from jax.experimental.pallas import tpu as pltpu
from jax.experimental.pallas import tpu_sc as plsc

MAXLEN = 200
EMBED = 32
VOCAB = 1000000
BATCH = 4096
ROWS = BATCH * MAXLEN

NC, NS = 2, 16
NW = NC * NS

NBLK = VOCAB // 128
TAIL = VOCAB - NBLK * 128
K1_EXTRA = NBLK - (NBLK // NW) * NW
K1_CNT = NBLK // NW


PITCH = 129


def _k1_body(tt_ref, tail_ref, rm_ref, bin_v, bout_v, isem0, isem1, ssem0, ssem1):
    wid = lax.axis_index("s") * NC + lax.axis_index("c")
    cnt = jnp.where(wid < K1_EXTRA, K1_CNT + 1, K1_CNT)
    start = wid * K1_CNT + jnp.minimum(wid, K1_EXTRA)
    iota = lax.iota(jnp.int32, 16)
    isems = (isem0, isem1)
    ssems = (ssem0, ssem1)

    def do_block(i, b):
        @pl.when(i < cnt)
        def _():
            j = start + i
            src = tt_ref.at[:, pl.ds(j * 128, 128)]
            dst = rm_ref.at[pl.ds(j * 32, 32)]
            bpad = bin_v.at[b, :, pl.ds(0, 128)]

            @pl.when(i >= 2)
            def _():
                pltpu.make_async_copy(bout_v.at[b], dst, ssems[b]).wait()
                pltpu.make_async_copy(src, bpad, isems[b]).wait()

            @pl.when(i < 2)
            def _():
                pltpu.sync_copy(src, bpad)

            def tstep(q, acc):
                vecs = []
                for tt in range(4):
                    bc = jnp.broadcast_to(q * 4 + tt, (16,))
                    for h in range(2):
                        vecs.append(
                            plsc.load_gather(bin_v.at[b], [iota + (h * 16), bc])
                        )
                for n, vec in enumerate(vecs):
                    bout_v[b, q, pl.ds(n * 16, 16)] = vec
                return acc

            lax.fori_loop(0, 32, tstep, 0)

            @pl.when(i + 2 < cnt)
            def _():
                nsrc = tt_ref.at[:, pl.ds((start + i + 2) * 128, 128)]
                pltpu.async_copy(nsrc, bpad, isems[b])

            pltpu.async_copy(bout_v.at[b], dst, ssems[b])

    def pair(k, carry):
        do_block(2 * k, 0)
        do_block(2 * k + 1, 1)
        return carry

    lax.fori_loop(0, (K1_CNT + 2) // 2, pair, 0)
    for b in range(2):
        pltpu.make_async_copy(
            bout_v.at[b], rm_ref.at[pl.ds(0, 32)], ssems[b]
        ).wait()

    @pl.when(wid == NW - 1)
    def _():
        pltpu.sync_copy(tail_ref, rm_ref.at[pl.ds(NBLK * 32, TAIL * EMBED // 128)])


K2_UNITS = ROWS // 1024
K2_PER_W = K2_UNITS // NW


def _k2_body(valt_ref, rm_ref, pat_ref, mid_ref, idx_v, buf_v, pat_v, sem, osem0, osem1):
    wid = lax.axis_index("s") * NC + lax.axis_index("c")
    pltpu.sync_copy(pat_ref, pat_v)
    base_u = wid * K2_PER_W

    def do_unit(i, b, osem):
        u = base_u + i
        s = u // 4
        prow = s // 4
        pcol = (s % 4) * EMBED
        pltpu.sync_copy(valt_ref.at[pl.ds(u * 8, 8)], idx_v.at[b])
        copies = []
        for j in range(8):
            copies.append(
                pltpu.async_copy(
                    rm_ref.at[idx_v.at[b, j]],
                    buf_v.at[b, pl.ds(j * 128, 128)],
                    sem,
                )
            )
        for cp in copies:
            cp.wait()
        pv0 = pat_v[prow, pl.ds(pcol, 16)]
        pv1 = pat_v[prow, pl.ds(pcol + 16, 16)]

        def add_step(q, acc):
            for rr in range(4):
                r = q * 4 + rr
                plsc.addupdate(buf_v.at[b, r, pl.ds(0, 16)], pv0)
                plsc.addupdate(buf_v.at[b, r, pl.ds(16, 16)], pv1)
            return acc

        lax.fori_loop(0, 256, add_step, 0)
        return pltpu.async_copy(buf_v.at[b], mid_ref.at[pl.ds(u * 1024, 1024)], osem)

    sems = [osem0, osem1]
    handles = [None, None]
    for i in range(K2_PER_W):
        b = i % 2
        if handles[b] is not None:
            handles[b].wait()
        handles[b] = do_unit(i, b, sems[b])
    handles[0].wait()
    handles[1].wait()


K3_UNITS = MAXLEN * (BATCH // 512)
K3_PER_W = K3_UNITS // NW


def _k3_body(mid_ref, out_ref, bin_v, bout_v, isem0, isem1, ssem0, ssem1):
    wid = lax.axis_index("s") * NC + lax.axis_index("c")
    base_u = wid * K3_PER_W
    iota = lax.iota(jnp.int32, 16)
    isems = (isem0, isem1)
    ssems = (ssem0, ssem1)

    def unit_src(u):
        return mid_ref.at[pl.ds(u * 128, 128)]

    def do_unit(i, b):
        u = base_u + i
        s = u // 8
        jcol = (u % 8) * 512

        @pl.when(i >= 2)
        def _():
            for jj in range(4):
                pltpu.make_async_copy(
                    bout_v.at[b, jj, :, pl.ds(0, 128)],
                    out_ref.at[s, :, pl.ds(jcol + jj * 128, 128)],
                    ssems[b],
                ).wait()
            pltpu.make_async_copy(unit_src(u), bin_v.at[b], isems[b]).wait()

        @pl.when(i < 2)
        def _():
            pltpu.sync_copy(unit_src(u), bin_v.at[b])

        for jj in range(4):
            def tstep(q, acc, jj=jj):
                vecs = []
                for n in range(8):
                    vecs.append(bin_v[b, jj * 32 + q, pl.ds(n * 16, 16)])
                for tt in range(4):
                    bc = jnp.broadcast_to(q * 4 + tt, (16,))
                    for h in range(2):
                        plsc.store_scatter(
                            bout_v.at[b, jj], [iota + (h * 16), bc],
                            vecs[tt * 2 + h],
                        )
                return acc

            lax.fori_loop(0, 32, tstep, 0)

        @pl.when(i + 2 < cnt_true)
        def _():
            pltpu.async_copy(unit_src(u + 2), bin_v.at[b], isems[b])

        for jj in range(4):
            pltpu.async_copy(
                bout_v.at[b, jj, :, pl.ds(0, 128)],
                out_ref.at[s, :, pl.ds(jcol + jj * 128, 128)],
                ssems[b],
            )

    cnt_true = K3_PER_W

    def pair(k, carry):
        do_unit(2 * k, 0)
        do_unit(2 * k + 1, 1)
        return carry

    lax.fori_loop(0, K3_PER_W // 2, pair, 0)
    for b in range(2):
        for jj in range(4):
            pltpu.make_async_copy(
                bout_v.at[b, jj, :, pl.ds(0, 128)],
                out_ref.at[0, :, pl.ds(jj * 128, 128)],
                ssems[b],
            ).wait()


_MESH = plsc.VectorSubcoreMesh(core_axis_name="c", subcore_axis_name="s")


@jax.jit
def _run(val, token_table, pos_table):
    tt = token_table.T
    tail = token_table[NBLK * 128:].reshape(TAIL * EMBED // 128, 128)
    valt = val.T.astype(jnp.int32).reshape(ROWS // 128, 128)
    patq = pos_table.reshape(MAXLEN * EMBED // 128, 128)

    k1 = functools.partial(
        pl.kernel,
        mesh=_MESH,
        out_type=jax.ShapeDtypeStruct((VOCAB * EMBED // 128, 128), jnp.float32),
        scratch_types=[
            pltpu.VMEM((2, EMBED, PITCH), jnp.float32),
            pltpu.VMEM((2, 32, 128), jnp.float32),
            pltpu.SemaphoreType.DMA,
            pltpu.SemaphoreType.DMA,
            pltpu.SemaphoreType.DMA,
            pltpu.SemaphoreType.DMA,
        ],
        compiler_params=pltpu.CompilerParams(needs_layout_passes=False),
    )(_k1_body)
    rm4 = k1(tt, tail)

    k2 = functools.partial(
        pl.kernel,
        mesh=_MESH,
        out_type=jax.ShapeDtypeStruct((ROWS, EMBED), jnp.float32),
        scratch_types=[
            pltpu.VMEM((2, 8, 128), jnp.int32),
            pltpu.VMEM((2, 1024, EMBED), jnp.float32),
            pltpu.VMEM((MAXLEN * EMBED // 128, 128), jnp.float32),
            pltpu.SemaphoreType.DMA,
            pltpu.SemaphoreType.DMA,
            pltpu.SemaphoreType.DMA,
        ],
        compiler_params=pltpu.CompilerParams(use_tc_tiling_on_sc=False),
    )(_k2_body)
    mid = k2(valt, rm4.reshape(VOCAB, EMBED), patq)

    k3 = functools.partial(
        pl.kernel,
        mesh=_MESH,
        out_type=jax.ShapeDtypeStruct((MAXLEN, EMBED, BATCH), jnp.float32),
        scratch_types=[
            pltpu.VMEM((2, 128, 128), jnp.float32),
            pltpu.VMEM((2, 4, EMBED, PITCH), jnp.float32),
            pltpu.SemaphoreType.DMA,
            pltpu.SemaphoreType.DMA,
            pltpu.SemaphoreType.DMA,
            pltpu.SemaphoreType.DMA,
        ],
        compiler_params=pltpu.CompilerParams(needs_layout_passes=False),
    )(_k3_body)
    outt = k3(mid.reshape(ROWS * EMBED // 128, 128))
    return jnp.transpose(outt, (2, 0, 1))


def kernel(val, token_table, pos_table):
    return _run(val, token_table, pos_table)
